# baseline (speedup 1.0000x reference)
import functools
import jax
import jax.numpy as jnp
from jax import lax
from jax.experimental import pallas as pl
from jax.experimental.pallas import tpu as pltpu
from jax.experimental.pallas import tpu_sc as plsc

N_NODES = 10000
D = 128
E_EDGES = 320000
ROW_BLK = 1000

NCB = 8
NQ = 4
EPQ = E_EDGES // NQ
CE = 640
NCHUNK = EPQ // CE
NH = 5008
NEG_INF = float("-inf")


def _segmax_body(xcb_hbm, src_hbm, dst_hbm, out_hbm,
                 acc, srcb, dstb, idxb, offb, rows, sem):
    cid = lax.axis_index("c")
    sid = lax.axis_index("s")
    wid = sid * 2 + cid
    cb = wid % NCB
    q = wid // NCB

    for p in range(2):
        @pl.loop(0, NH + 16)
        def _init(i):
            acc[i, :] = jnp.full((16,), NEG_INF, jnp.float32)

        @pl.loop(0, NCHUNK)
        def _chunk(c):
            base_e = q * EPQ + c * CE
            pltpu.sync_copy(src_hbm.at[pl.ds(base_e, CE)], srcb)
            pltpu.sync_copy(dst_hbm.at[pl.ds(base_e, CE)], dstb)

            cbase = jnp.int32(cb * N_NODES)
            pbase = jnp.int32(p * NH)

            lane = lax.iota(jnp.int32, 16)

            @pl.loop(0, CE // 16)
            def _mkidx(i):
                sl = pl.ds(i * 16, 16)
                idxb[sl] = srcb[sl] + cbase
                dv = dstb[sl] - pbase
                ok = (dv >= 0) & (dv < NH)
                offb[sl] = jnp.where(ok, dv, NH + lane)

            copies = [
                pltpu.async_copy(
                    xcb_hbm.at[idxb.at[pl.ds(g * 128, 128)]],
                    rows.at[pl.ds(g * 128, 128), :], sem)
                for g in range(CE // 128)
            ]
            for cp in copies:
                cp.wait()

            shift_pat = jnp.maximum(lane - 1, 0)
            notfirst = lane > 0

            @pl.loop(0, CE // 16)
            def _rmw(i):
                offv = offb[pl.ds(i * 16, 16)]
                rowv = lane + i * 16
                sk, _sv = plsc.sort_key_val(offv, offv)
                dup = (sk == sk[shift_pat]) & notfirst
                ndups = plsc.all_reduce_population_count(dup)

                @pl.when(ndups[0] == 0)
                def _fast():
                    for c in range(16):
                        colv = jnp.full((16,), c, jnp.int32)
                        av = plsc.load_gather(acc, [offv, colv])
                        rv = plsc.load_gather(rows, [rowv, colv])
                        plsc.store_scatter(acc, [offv, colv],
                                           jnp.maximum(av, rv))

                @pl.when(ndups[0] > 0)
                def _slow():
                    for jj in range(16):
                        r = offv[jj]

                        @pl.when(r < NH)
                        def _():
                            acc[r, :] = jnp.maximum(acc[r, :],
                                                    rows[i * 16 + jj, :])

        nrows = NH if p == 0 else (N_NODES - NH)
        pltpu.sync_copy(
            acc.at[pl.ds(0, nrows), :],
            out_hbm.at[q].at[pl.ds(cb * N_NODES + p * NH, nrows), :])


def _segmax_sc(xcb, src, dst):
    mesh = plsc.VectorSubcoreMesh(core_axis_name="c", subcore_axis_name="s")
    kern = pl.kernel(
        _segmax_body,
        out_type=jax.ShapeDtypeStruct((NQ, NCB * N_NODES, 16), jnp.float32),
        mesh=mesh,
        compiler_params=pltpu.CompilerParams(use_tc_tiling_on_sc=False,
                                             needs_layout_passes=False),
        scratch_types=[
            pltpu.VMEM((NH + 16, 16), jnp.float32),
            pltpu.VMEM((CE,), jnp.int32),
            pltpu.VMEM((CE,), jnp.int32),
            pltpu.VMEM((CE,), jnp.int32),
            pltpu.VMEM((CE,), jnp.int32),
            pltpu.VMEM((CE, 16), jnp.float32),
            pltpu.SemaphoreType.DMA,
        ],
    )
    return kern(xcb, src, dst)


def _to_cb(a):
    return a.reshape(N_NODES, NCB, 16).transpose(1, 0, 2).reshape(
        NCB * N_NODES, 16)


def _parts_std(p):
    return p.reshape(NQ, NCB, N_NODES, 16).transpose(0, 2, 1, 3).reshape(
        NQ, N_NODES, D)


def _merge(p_ref):
    agg = jnp.max(p_ref[...], axis=0)
    return jnp.where(agg == NEG_INF, jnp.float32(0.0), agg)


def _dense1_body(x_ref, p_ref, w_ref, b_ref, lnw_ref, lnb_ref, eps_ref,
                 a_ref, o_ref):
    agg = _merge(p_ref)
    h = (1.0 + eps_ref[0, 0]) * x_ref[...] + agg
    h = jnp.dot(h, w_ref[...], preferred_element_type=jnp.float32) + b_ref[...]
    mu = jnp.mean(h, axis=-1, keepdims=True)
    var = jnp.mean((h - mu) ** 2, axis=-1, keepdims=True)
    h = (h - mu) * jax.lax.rsqrt(var + 1e-5) * lnw_ref[...] + lnb_ref[...]
    o_ref[...] = jnp.where(h > 0, h, a_ref[0, 0] * h)


def _dense2_body(h_ref, p_ref, w_ref, b_ref, eps_ref, o_ref):
    t = (1.0 + eps_ref[0, 0]) * h_ref[...] + _merge(p_ref)
    o_ref[...] = jnp.dot(t, w_ref[...], preferred_element_type=jnp.float32) \
        + b_ref[...]


def _dense1(x, parts, W1T, b1, ln_w, ln_b, eps1, prelu_a):
    grid = (N_NODES // ROW_BLK,)
    blk = pl.BlockSpec((ROW_BLK, D), lambda i: (i, 0))
    pblk = pl.BlockSpec((NQ, ROW_BLK, D), lambda i: (0, i, 0))
    full = pl.BlockSpec((D, D), lambda i: (0, 0))
    vec = pl.BlockSpec((1, D), lambda i: (0, 0))
    sca = pl.BlockSpec((1, 1), lambda i: (0, 0))
    return pl.pallas_call(
        _dense1_body,
        grid=grid,
        in_specs=[blk, pblk, full, vec, vec, vec, sca, sca],
        out_specs=blk,
        out_shape=jax.ShapeDtypeStruct((N_NODES, D), jnp.float32),
    )(x, parts, W1T, b1.reshape(1, D), ln_w.reshape(1, D), ln_b.reshape(1, D),
      eps1.reshape(1, 1), prelu_a.reshape(1, 1))


def _dense2(h, parts, W2T, b2, eps2):
    grid = (N_NODES // ROW_BLK,)
    blk = pl.BlockSpec((ROW_BLK, D), lambda i: (i, 0))
    pblk = pl.BlockSpec((NQ, ROW_BLK, D), lambda i: (0, i, 0))
    full = pl.BlockSpec((D, D), lambda i: (0, 0))
    vec = pl.BlockSpec((1, D), lambda i: (0, 0))
    sca = pl.BlockSpec((1, 1), lambda i: (0, 0))
    return pl.pallas_call(
        _dense2_body,
        grid=grid,
        in_specs=[blk, pblk, full, vec, sca],
        out_specs=blk,
        out_shape=jax.ShapeDtypeStruct((N_NODES, D), jnp.float32),
    )(h, parts, W2T, b2.reshape(1, D), eps2.reshape(1, 1))


@jax.jit
def kernel(x, edge_index, W1, b1, eps1, ln_w, ln_b, prelu_a, W2, b2, eps2):
    src = edge_index[0]
    dst = edge_index[1]
    p1 = _parts_std(_segmax_sc(_to_cb(x), src, dst))
    h = _dense1(x, p1, W1.T, b1, ln_w, ln_b, eps1, prelu_a)
    p2 = _parts_std(_segmax_sc(_to_cb(h), src, dst))
    return _dense2(h, p2, W2.T, b2, eps2)


# speedup vs baseline: 4.0666x; 4.0666x over previous
import functools
import jax
import jax.numpy as jnp
from jax import lax
from jax.experimental import pallas as pl
from jax.experimental.pallas import tpu as pltpu
from jax.experimental.pallas import tpu_sc as plsc

N_NODES = 10000
D = 128
E_EDGES = 320000
ROW_BLK = 1000

NW = 32
RNG = 313
NPAD = NW * RNG
CE = 2560
NCHUNK = E_EDGES // CE
NSUB = CE // 128
GB = 256
QCAP = 384
NEG_INF = float("-inf")


def _drain_batch(x_hbm, qsrc, qdst, rowsb, acc):
    for gg in range(GB // 128):
        pltpu.sync_copy(
            x_hbm.at[qsrc.at[pl.ds(gg * 128, 128)]],
            rowsb.at[pl.ds(gg * 128, 128), :])

    @pl.loop(0, GB // 16)
    def _rmw(i):
        offv = qdst[pl.ds(i * 16, 16)]
        for jj in range(16):
            r = offv[jj]
            for cc in range(8):
                sl = pl.ds(cc * 16, 16)
                acc[r, sl] = jnp.maximum(acc[r, sl], rowsb[i * 16 + jj, sl])


def _segmax_body(x_hbm, src_hbm, dst_hbm, out_hbm,
                 acc, srcb, dstb, qsrc, qdst, rowsb, sem):
    cid = lax.axis_index("c")
    sid = lax.axis_index("s")
    wid = sid * 2 + cid
    lo = wid * RNG
    lane = lax.iota(jnp.int32, 16)

    @pl.loop(0, RNG + 1)
    def _init(i):
        for cc in range(8):
            acc[i, pl.ds(cc * 16, 16)] = jnp.full((16,), NEG_INF, jnp.float32)

    def sub_body(s, qlen):
        stats = []
        for g in range(8):
            sl = pl.ds(s * 128 + g * 16, 16)
            dloc = dstb[sl] - lo
            m = (dloc >= 0) & (dloc < RNG)
            mi = m.astype(jnp.int32)
            cum = plsc.cumsum(mi)
            stats.append((sl, m, mi, cum, dloc))
        qb = qlen
        for (sl, m, mi, cum, dloc) in stats:
            addr = (cum - mi) + qb
            plsc.store_scatter(qsrc, [addr], srcb[sl], mask=m)
            plsc.store_scatter(qdst, [addr], dloc, mask=m)
            qb = qb + cum[15]
        drained = qb >= GB

        @pl.when(drained)
        def _():
            _drain_batch(x_hbm, qsrc, qdst, rowsb, acc)
            for t in range(8):
                tsl = pl.ds(t * 16, 16)
                ssl = pl.ds(GB + t * 16, 16)
                qsrc[tsl] = qsrc[ssl]
                qdst[tsl] = qdst[ssl]

        return jnp.where(drained, qb - GB, qb)

    def chunk_body(c, qlen):
        pltpu.sync_copy(src_hbm.at[pl.ds(c * CE, CE)], srcb)
        pltpu.sync_copy(dst_hbm.at[pl.ds(c * CE, CE)], dstb)
        return lax.fori_loop(0, NSUB, sub_body, qlen)

    qlen = lax.fori_loop(0, NCHUNK, chunk_body, jnp.int32(0))

    for t in range(GB // 16):
        sl = pl.ds(t * 16, 16)
        mpad = (lane + t * 16) < qlen
        qsrc[sl] = jnp.where(mpad, qsrc[sl], jnp.int32(0))
        qdst[sl] = jnp.where(mpad, qdst[sl], jnp.int32(RNG))
    _drain_batch(x_hbm, qsrc, qdst, rowsb, acc)

    @pl.loop(0, RNG)
    def _fin(i):
        for cc in range(8):
            sl = pl.ds(cc * 16, 16)
            v = acc[i, sl]
            acc[i, sl] = jnp.where(v == NEG_INF, jnp.float32(0.0), v)

    pltpu.sync_copy(acc.at[pl.ds(0, RNG), :],
                    out_hbm.at[pl.ds(lo, RNG), :])


def _segmax_sc(x, src, dst):
    mesh = plsc.VectorSubcoreMesh(core_axis_name="c", subcore_axis_name="s")
    kern = pl.kernel(
        _segmax_body,
        out_type=jax.ShapeDtypeStruct((NPAD, D), jnp.float32),
        mesh=mesh,
        compiler_params=pltpu.CompilerParams(use_tc_tiling_on_sc=False,
                                             needs_layout_passes=False),
        scratch_types=[
            pltpu.VMEM((RNG + 1, D), jnp.float32),
            pltpu.VMEM((CE,), jnp.int32),
            pltpu.VMEM((CE,), jnp.int32),
            pltpu.VMEM((QCAP,), jnp.int32),
            pltpu.VMEM((QCAP,), jnp.int32),
            pltpu.VMEM((GB, D), jnp.float32),
            pltpu.SemaphoreType.DMA,
        ],
    )
    return kern(x, src, dst)[:N_NODES]


def _dense1_body(x_ref, agg_ref, w_ref, b_ref, lnw_ref, lnb_ref, eps_ref,
                 a_ref, o_ref):
    h = (1.0 + eps_ref[0, 0]) * x_ref[...] + agg_ref[...]
    h = jnp.dot(h, w_ref[...], preferred_element_type=jnp.float32) + b_ref[...]
    mu = jnp.mean(h, axis=-1, keepdims=True)
    var = jnp.mean((h - mu) ** 2, axis=-1, keepdims=True)
    h = (h - mu) * jax.lax.rsqrt(var + 1e-5) * lnw_ref[...] + lnb_ref[...]
    o_ref[...] = jnp.where(h > 0, h, a_ref[0, 0] * h)


def _dense2_body(h_ref, agg_ref, w_ref, b_ref, eps_ref, o_ref):
    t = (1.0 + eps_ref[0, 0]) * h_ref[...] + agg_ref[...]
    o_ref[...] = jnp.dot(t, w_ref[...], preferred_element_type=jnp.float32) \
        + b_ref[...]


def _dense1(x, agg, W1T, b1, ln_w, ln_b, eps1, prelu_a):
    grid = (N_NODES // ROW_BLK,)
    blk = pl.BlockSpec((ROW_BLK, D), lambda i: (i, 0))
    full = pl.BlockSpec((D, D), lambda i: (0, 0))
    vec = pl.BlockSpec((1, D), lambda i: (0, 0))
    sca = pl.BlockSpec((1, 1), lambda i: (0, 0))
    return pl.pallas_call(
        _dense1_body,
        grid=grid,
        in_specs=[blk, blk, full, vec, vec, vec, sca, sca],
        out_specs=blk,
        out_shape=jax.ShapeDtypeStruct((N_NODES, D), jnp.float32),
    )(x, agg, W1T, b1.reshape(1, D), ln_w.reshape(1, D), ln_b.reshape(1, D),
      eps1.reshape(1, 1), prelu_a.reshape(1, 1))


def _dense2(h, agg, W2T, b2, eps2):
    grid = (N_NODES // ROW_BLK,)
    blk = pl.BlockSpec((ROW_BLK, D), lambda i: (i, 0))
    full = pl.BlockSpec((D, D), lambda i: (0, 0))
    vec = pl.BlockSpec((1, D), lambda i: (0, 0))
    sca = pl.BlockSpec((1, 1), lambda i: (0, 0))
    return pl.pallas_call(
        _dense2_body,
        grid=grid,
        in_specs=[blk, blk, full, vec, sca],
        out_specs=blk,
        out_shape=jax.ShapeDtypeStruct((N_NODES, D), jnp.float32),
    )(h, agg, W2T, b2.reshape(1, D), eps2.reshape(1, 1))


@jax.jit
def kernel(x, edge_index, W1, b1, eps1, ln_w, ln_b, prelu_a, W2, b2, eps2):
    src = edge_index[0]
    dst = edge_index[1]
    agg1 = _segmax_sc(x, src, dst)
    h = _dense1(x, agg1, W1.T, b1, ln_w, ln_b, eps1, prelu_a)
    agg2 = _segmax_sc(h, src, dst)
    return _dense2(h, agg2, W2.T, b2, eps2)


# speedup vs baseline: 5.0643x; 1.2453x over previous
import functools
import jax
import jax.numpy as jnp
from jax import lax
from jax.experimental import pallas as pl
from jax.experimental.pallas import tpu as pltpu
from jax.experimental.pallas import tpu_sc as plsc

N_NODES = 10000
D = 128
E_EDGES = 320000
ROW_BLK = 1000

NW = 32
RNG = 313
NPAD = NW * RNG
CE = 2560
NCHUNK = E_EDGES // CE
NSUB = CE // 128
GB = 256
QCAP = 384
NEG_INF = float("-inf")


def _drain_batch(x_hbm, qsrc, qdst, rowsb, acc, semg):
    cps = [
        pltpu.async_copy(
            x_hbm.at[qsrc.at[pl.ds(gg * 128, 128)]],
            rowsb.at[pl.ds(gg * 128, 128), :], semg)
        for gg in range(GB // 128)
    ]
    for cp in cps:
        cp.wait()

    @pl.loop(0, GB // 16)
    def _rmw(i):
        offv = qdst[pl.ds(i * 16, 16)]
        for jj in range(16):
            r = offv[jj]
            for cc in range(8):
                sl = pl.ds(cc * 16, 16)
                acc[r, sl] = jnp.maximum(acc[r, sl], rowsb[i * 16 + jj, sl])


def _segmax_body(x_hbm, src_hbm, dst_hbm, out_hbm,
                 acc, srcb, dstb, qsrc, qdst, rowsb, sem0, sem1, semg):
    cid = lax.axis_index("c")
    sid = lax.axis_index("s")
    wid = sid * 2 + cid
    lo = wid * RNG
    lane = lax.iota(jnp.int32, 16)

    @pl.loop(0, RNG + 1)
    def _init(i):
        for cc in range(8):
            acc[i, pl.ds(cc * 16, 16)] = jnp.full((16,), NEG_INF, jnp.float32)

    def scan_sub(b, s, qlen):
        stats = []
        for g in range(8):
            sl = pl.ds(s * 128 + g * 16, 16)
            dloc = dstb[b, sl] - lo
            m = (dloc >= 0) & (dloc < RNG)
            mi = m.astype(jnp.int32)
            cum = plsc.cumsum(mi)
            stats.append((sl, m, mi, cum, dloc))
        qb = qlen
        for (sl, m, mi, cum, dloc) in stats:
            addr = (cum - mi) + qb
            plsc.store_scatter(qsrc, [addr], srcb[b, sl], mask=m)
            plsc.store_scatter(qdst, [addr], dloc, mask=m)
            qb = qb + cum[15]
        drained = qb >= GB

        @pl.when(drained)
        def _():
            _drain_batch(x_hbm, qsrc, qdst, rowsb, acc, semg)
            for t in range(8):
                tsl = pl.ds(t * 16, 16)
                ssl = pl.ds(GB + t * 16, 16)
                qsrc[tsl] = qsrc[ssl]
                qdst[tsl] = qdst[ssl]

        return jnp.where(drained, qb - GB, qb)

    def scan_chunk(b, qlen):
        return lax.fori_loop(0, NSUB, functools.partial(scan_sub, b), qlen)

    def issue(c, b, sem):
        pltpu.async_copy(src_hbm.at[pl.ds(c * CE, CE)], srcb.at[b], sem)
        pltpu.async_copy(dst_hbm.at[pl.ds(c * CE, CE)], dstb.at[b], sem)

    def wait(c, b, sem):
        pltpu.make_async_copy(
            src_hbm.at[pl.ds(c * CE, CE)], srcb.at[b], sem).wait()
        pltpu.make_async_copy(
            dst_hbm.at[pl.ds(c * CE, CE)], dstb.at[b], sem).wait()

    HALF = NCHUNK // 2
    issue(0, 0, sem0)

    def pair_body(cp, qlen):
        c0 = 2 * cp
        issue(c0 + 1, 1, sem1)
        wait(c0, 0, sem0)
        qlen = scan_chunk(0, qlen)

        @pl.when(cp < HALF - 1)
        def _():
            issue(c0 + 2, 0, sem0)

        wait(c0 + 1, 1, sem1)
        return scan_chunk(1, qlen)

    qlen = lax.fori_loop(0, HALF, pair_body, jnp.int32(0))
    if NCHUNK % 2:
        c = NCHUNK - 1
        pltpu.sync_copy(src_hbm.at[pl.ds(c * CE, CE)], srcb.at[0])
        pltpu.sync_copy(dst_hbm.at[pl.ds(c * CE, CE)], dstb.at[0])
        qlen = scan_chunk(0, qlen)

    for t in range(GB // 16):
        sl = pl.ds(t * 16, 16)
        mpad = (lane + t * 16) < qlen
        qsrc[sl] = jnp.where(mpad, qsrc[sl], jnp.int32(0))
        qdst[sl] = jnp.where(mpad, qdst[sl], jnp.int32(RNG))
    _drain_batch(x_hbm, qsrc, qdst, rowsb, acc, semg)

    @pl.loop(0, RNG)
    def _fin(i):
        for cc in range(8):
            sl = pl.ds(cc * 16, 16)
            v = acc[i, sl]
            acc[i, sl] = jnp.where(v == NEG_INF, jnp.float32(0.0), v)

    pltpu.sync_copy(acc.at[pl.ds(0, RNG), :],
                    out_hbm.at[pl.ds(lo, RNG), :])


def _segmax_sc(x, src, dst):
    mesh = plsc.VectorSubcoreMesh(core_axis_name="c", subcore_axis_name="s")
    kern = pl.kernel(
        _segmax_body,
        out_type=jax.ShapeDtypeStruct((NPAD, D), jnp.float32),
        mesh=mesh,
        compiler_params=pltpu.CompilerParams(use_tc_tiling_on_sc=False,
                                             needs_layout_passes=False),
        scratch_types=[
            pltpu.VMEM((RNG + 1, D), jnp.float32),
            pltpu.VMEM((2, CE), jnp.int32),
            pltpu.VMEM((2, CE), jnp.int32),
            pltpu.VMEM((QCAP,), jnp.int32),
            pltpu.VMEM((QCAP,), jnp.int32),
            pltpu.VMEM((GB, D), jnp.float32),
            pltpu.SemaphoreType.DMA,
            pltpu.SemaphoreType.DMA,
            pltpu.SemaphoreType.DMA,
        ],
    )
    return kern(x, src, dst)[:N_NODES]


def _dense1_body(x_ref, agg_ref, w_ref, b_ref, lnw_ref, lnb_ref, eps_ref,
                 a_ref, o_ref):
    h = (1.0 + eps_ref[0, 0]) * x_ref[...] + agg_ref[...]
    h = jnp.dot(h, w_ref[...], preferred_element_type=jnp.float32) + b_ref[...]
    mu = jnp.mean(h, axis=-1, keepdims=True)
    var = jnp.mean((h - mu) ** 2, axis=-1, keepdims=True)
    h = (h - mu) * jax.lax.rsqrt(var + 1e-5) * lnw_ref[...] + lnb_ref[...]
    o_ref[...] = jnp.where(h > 0, h, a_ref[0, 0] * h)


def _dense2_body(h_ref, agg_ref, w_ref, b_ref, eps_ref, o_ref):
    t = (1.0 + eps_ref[0, 0]) * h_ref[...] + agg_ref[...]
    o_ref[...] = jnp.dot(t, w_ref[...], preferred_element_type=jnp.float32) \
        + b_ref[...]


def _dense1(x, agg, W1T, b1, ln_w, ln_b, eps1, prelu_a):
    grid = (N_NODES // ROW_BLK,)
    blk = pl.BlockSpec((ROW_BLK, D), lambda i: (i, 0))
    full = pl.BlockSpec((D, D), lambda i: (0, 0))
    vec = pl.BlockSpec((1, D), lambda i: (0, 0))
    sca = pl.BlockSpec((1, 1), lambda i: (0, 0))
    return pl.pallas_call(
        _dense1_body,
        grid=grid,
        in_specs=[blk, blk, full, vec, vec, vec, sca, sca],
        out_specs=blk,
        out_shape=jax.ShapeDtypeStruct((N_NODES, D), jnp.float32),
    )(x, agg, W1T, b1.reshape(1, D), ln_w.reshape(1, D), ln_b.reshape(1, D),
      eps1.reshape(1, 1), prelu_a.reshape(1, 1))


def _dense2(h, agg, W2T, b2, eps2):
    grid = (N_NODES // ROW_BLK,)
    blk = pl.BlockSpec((ROW_BLK, D), lambda i: (i, 0))
    full = pl.BlockSpec((D, D), lambda i: (0, 0))
    vec = pl.BlockSpec((1, D), lambda i: (0, 0))
    sca = pl.BlockSpec((1, 1), lambda i: (0, 0))
    return pl.pallas_call(
        _dense2_body,
        grid=grid,
        in_specs=[blk, blk, full, vec, sca],
        out_specs=blk,
        out_shape=jax.ShapeDtypeStruct((N_NODES, D), jnp.float32),
    )(h, agg, W2T, b2.reshape(1, D), eps2.reshape(1, 1))


@jax.jit
def kernel(x, edge_index, W1, b1, eps1, ln_w, ln_b, prelu_a, W2, b2, eps2):
    src = edge_index[0]
    dst = edge_index[1]
    agg1 = _segmax_sc(x, src, dst)
    h = _dense1(x, agg1, W1.T, b1, ln_w, ln_b, eps1, prelu_a)
    agg2 = _segmax_sc(h, src, dst)
    return _dense2(h, agg2, W2.T, b2, eps2)


# speedup vs baseline: 5.2187x; 1.0305x over previous
import functools
import jax
import jax.numpy as jnp
from jax import lax
from jax.experimental import pallas as pl
from jax.experimental.pallas import tpu as pltpu
from jax.experimental.pallas import tpu_sc as plsc

N_NODES = 10000
D = 128
E_EDGES = 320000
ROW_BLK = 1000

NW = 32
RNG = 313
NPAD = NW * RNG
CE = 2560
NCHUNK = E_EDGES // CE
NSUB = CE // 128
GB = 256
QCAP = 384
NEG_INF = float("-inf")


def _drain_batch(x_hbm, qsrc, qdst, rowsb, acc, semg):
    cps = [
        pltpu.async_copy(
            x_hbm.at[qsrc.at[pl.ds(gg * 128, 128)]],
            rowsb.at[pl.ds(gg * 128, 128), :], semg)
        for gg in range(GB // 128)
    ]

    for gg, cp in enumerate(cps):
        cp.wait()

        @pl.loop(gg * 8, (gg + 1) * 8)
        def _rmw(i):
            offv = qdst[pl.ds(i * 16, 16)]
            for jj in range(16):
                r = offv[jj]
                for cc in range(8):
                    sl = pl.ds(cc * 16, 16)
                    acc[r, sl] = jnp.maximum(acc[r, sl],
                                             rowsb[i * 16 + jj, sl])


def _segmax_body(x_hbm, src_hbm, dst_hbm, out_hbm,
                 acc, srcb, dstb, qsrc, qdst, rowsb, sem0, sem1, semg):
    cid = lax.axis_index("c")
    sid = lax.axis_index("s")
    wid = sid * 2 + cid
    lo = wid * RNG
    lane = lax.iota(jnp.int32, 16)

    @pl.loop(0, RNG + 1)
    def _init(i):
        for cc in range(8):
            acc[i, pl.ds(cc * 16, 16)] = jnp.full((16,), NEG_INF, jnp.float32)

    def scan_sub(b, s, qlen):
        stats = []
        for g in range(8):
            sl = pl.ds(s * 128 + g * 16, 16)
            dloc = dstb[b, sl] - lo
            m = (dloc >= 0) & (dloc < RNG)
            mi = m.astype(jnp.int32)
            cum = plsc.cumsum(mi)
            stats.append((sl, m, mi, cum, dloc))
        cnts = [cum[15] for (_, _, _, cum, _) in stats]
        qbs = [qlen]
        for cnt in cnts:
            qbs.append(qbs[-1] + cnt)
        qb = qbs[-1]
        for (sl, m, mi, cum, dloc), base in zip(stats, qbs):
            addr = (cum - mi) + base
            plsc.store_scatter(qsrc, [addr], srcb[b, sl], mask=m)
            plsc.store_scatter(qdst, [addr], dloc, mask=m)
        drained = qb >= GB

        @pl.when(drained)
        def _():
            _drain_batch(x_hbm, qsrc, qdst, rowsb, acc, semg)
            for t in range(8):
                tsl = pl.ds(t * 16, 16)
                ssl = pl.ds(GB + t * 16, 16)
                qsrc[tsl] = qsrc[ssl]
                qdst[tsl] = qdst[ssl]

        return jnp.where(drained, qb - GB, qb)

    def scan_chunk(b, qlen):
        return lax.fori_loop(0, NSUB, functools.partial(scan_sub, b), qlen)

    def issue(c, b, sem):
        pltpu.async_copy(src_hbm.at[pl.ds(c * CE, CE)], srcb.at[b], sem)
        pltpu.async_copy(dst_hbm.at[pl.ds(c * CE, CE)], dstb.at[b], sem)

    def wait(c, b, sem):
        pltpu.make_async_copy(
            src_hbm.at[pl.ds(c * CE, CE)], srcb.at[b], sem).wait()
        pltpu.make_async_copy(
            dst_hbm.at[pl.ds(c * CE, CE)], dstb.at[b], sem).wait()

    HALF = NCHUNK // 2
    issue(0, 0, sem0)

    def pair_body(cp, qlen):
        c0 = 2 * cp
        issue(c0 + 1, 1, sem1)
        wait(c0, 0, sem0)
        qlen = scan_chunk(0, qlen)

        @pl.when(cp < HALF - 1)
        def _():
            issue(c0 + 2, 0, sem0)

        wait(c0 + 1, 1, sem1)
        return scan_chunk(1, qlen)

    qlen = lax.fori_loop(0, HALF, pair_body, jnp.int32(0))
    if NCHUNK % 2:
        c = NCHUNK - 1
        pltpu.sync_copy(src_hbm.at[pl.ds(c * CE, CE)], srcb.at[0])
        pltpu.sync_copy(dst_hbm.at[pl.ds(c * CE, CE)], dstb.at[0])
        qlen = scan_chunk(0, qlen)

    for t in range(GB // 16):
        sl = pl.ds(t * 16, 16)
        mpad = (lane + t * 16) < qlen
        qsrc[sl] = jnp.where(mpad, qsrc[sl], jnp.int32(0))
        qdst[sl] = jnp.where(mpad, qdst[sl], jnp.int32(RNG))
    _drain_batch(x_hbm, qsrc, qdst, rowsb, acc, semg)

    @pl.loop(0, RNG)
    def _fin(i):
        for cc in range(8):
            sl = pl.ds(cc * 16, 16)
            v = acc[i, sl]
            acc[i, sl] = jnp.where(v == NEG_INF, jnp.float32(0.0), v)

    pltpu.sync_copy(acc.at[pl.ds(0, RNG), :],
                    out_hbm.at[pl.ds(lo, RNG), :])


def _segmax_sc(x, src, dst):
    mesh = plsc.VectorSubcoreMesh(core_axis_name="c", subcore_axis_name="s")
    kern = pl.kernel(
        _segmax_body,
        out_type=jax.ShapeDtypeStruct((NPAD, D), jnp.float32),
        mesh=mesh,
        compiler_params=pltpu.CompilerParams(use_tc_tiling_on_sc=False,
                                             needs_layout_passes=False),
        scratch_types=[
            pltpu.VMEM((RNG + 1, D), jnp.float32),
            pltpu.VMEM((2, CE), jnp.int32),
            pltpu.VMEM((2, CE), jnp.int32),
            pltpu.VMEM((QCAP,), jnp.int32),
            pltpu.VMEM((QCAP,), jnp.int32),
            pltpu.VMEM((GB, D), jnp.float32),
            pltpu.SemaphoreType.DMA,
            pltpu.SemaphoreType.DMA,
            pltpu.SemaphoreType.DMA,
        ],
    )
    return kern(x, src, dst)[:N_NODES]


def _dense1_body(x_ref, agg_ref, w_ref, b_ref, lnw_ref, lnb_ref, eps_ref,
                 a_ref, o_ref):
    h = (1.0 + eps_ref[0, 0]) * x_ref[...] + agg_ref[...]
    h = jnp.dot(h, w_ref[...], preferred_element_type=jnp.float32) + b_ref[...]
    mu = jnp.mean(h, axis=-1, keepdims=True)
    var = jnp.mean((h - mu) ** 2, axis=-1, keepdims=True)
    h = (h - mu) * jax.lax.rsqrt(var + 1e-5) * lnw_ref[...] + lnb_ref[...]
    o_ref[...] = jnp.where(h > 0, h, a_ref[0, 0] * h)


def _dense2_body(h_ref, agg_ref, w_ref, b_ref, eps_ref, o_ref):
    t = (1.0 + eps_ref[0, 0]) * h_ref[...] + agg_ref[...]
    o_ref[...] = jnp.dot(t, w_ref[...], preferred_element_type=jnp.float32) \
        + b_ref[...]


def _dense1(x, agg, W1T, b1, ln_w, ln_b, eps1, prelu_a):
    grid = (N_NODES // ROW_BLK,)
    blk = pl.BlockSpec((ROW_BLK, D), lambda i: (i, 0))
    full = pl.BlockSpec((D, D), lambda i: (0, 0))
    vec = pl.BlockSpec((1, D), lambda i: (0, 0))
    sca = pl.BlockSpec((1, 1), lambda i: (0, 0))
    return pl.pallas_call(
        _dense1_body,
        grid=grid,
        in_specs=[blk, blk, full, vec, vec, vec, sca, sca],
        out_specs=blk,
        out_shape=jax.ShapeDtypeStruct((N_NODES, D), jnp.float32),
    )(x, agg, W1T, b1.reshape(1, D), ln_w.reshape(1, D), ln_b.reshape(1, D),
      eps1.reshape(1, 1), prelu_a.reshape(1, 1))


def _dense2(h, agg, W2T, b2, eps2):
    grid = (N_NODES // ROW_BLK,)
    blk = pl.BlockSpec((ROW_BLK, D), lambda i: (i, 0))
    full = pl.BlockSpec((D, D), lambda i: (0, 0))
    vec = pl.BlockSpec((1, D), lambda i: (0, 0))
    sca = pl.BlockSpec((1, 1), lambda i: (0, 0))
    return pl.pallas_call(
        _dense2_body,
        grid=grid,
        in_specs=[blk, blk, full, vec, sca],
        out_specs=blk,
        out_shape=jax.ShapeDtypeStruct((N_NODES, D), jnp.float32),
    )(h, agg, W2T, b2.reshape(1, D), eps2.reshape(1, 1))


@jax.jit
def kernel(x, edge_index, W1, b1, eps1, ln_w, ln_b, prelu_a, W2, b2, eps2):
    src = edge_index[0]
    dst = edge_index[1]
    agg1 = _segmax_sc(x, src, dst)
    h = _dense1(x, agg1, W1.T, b1, ln_w, ln_b, eps1, prelu_a)
    agg2 = _segmax_sc(h, src, dst)
    return _dense2(h, agg2, W2.T, b2, eps2)
